# Initial kernel scaffold; baseline (speedup 1.0000x reference)
import jax
import jax.numpy as jnp
from jax.experimental import pallas as pl


def kernel(x, pe_table):
    raise NotImplementedError("write your pallas kernel here")


# speedup vs baseline: 1.9335x; 1.9335x over previous
import jax
import jax.numpy as jnp
from jax.experimental import pallas as pl

BATCH = 4
EMB_DIM = 1024
SEQ_LEN = 8192

D_BLK = 256
S_BLK = 1024


def _body(x_ref, pe_ref, o_ref):
    pe_t = jnp.transpose(pe_ref[...], (1, 0))
    o_ref[...] = x_ref[...] + pe_t[None, :, :]


def kernel(x, pe_table):
    grid = (EMB_DIM // D_BLK, SEQ_LEN // S_BLK)
    return pl.pallas_call(
        _body,
        grid=grid,
        in_specs=[
            pl.BlockSpec((BATCH, D_BLK, S_BLK), lambda i, j: (0, i, j)),
            pl.BlockSpec((S_BLK, D_BLK), lambda i, j: (j, i)),
        ],
        out_specs=pl.BlockSpec((BATCH, D_BLK, S_BLK), lambda i, j: (0, i, j)),
        out_shape=jax.ShapeDtypeStruct((BATCH, EMB_DIM, SEQ_LEN), jnp.float32),
    )(x, pe_table)


# speedup vs baseline: 1.9651x; 1.0163x over previous
import jax
import jax.numpy as jnp
from jax.experimental import pallas as pl

BATCH = 4
EMB_DIM = 1024
SEQ_LEN = 8192

D_BLK = 512
S_BLK = 1024


def _body(x_ref, pe_ref, o_ref):
    pe_t = jnp.transpose(pe_ref[...], (1, 0))
    o_ref[...] = x_ref[...] + pe_t[None, :, :]


def kernel(x, pe_table):
    grid = (EMB_DIM // D_BLK, SEQ_LEN // S_BLK)
    return pl.pallas_call(
        _body,
        grid=grid,
        in_specs=[
            pl.BlockSpec((BATCH, D_BLK, S_BLK), lambda i, j: (0, i, j)),
            pl.BlockSpec((S_BLK, D_BLK), lambda i, j: (j, i)),
        ],
        out_specs=pl.BlockSpec((BATCH, D_BLK, S_BLK), lambda i, j: (0, i, j)),
        out_shape=jax.ShapeDtypeStruct((BATCH, EMB_DIM, SEQ_LEN), jnp.float32),
    )(x, pe_table)


# speedup vs baseline: 1.9861x; 1.0107x over previous
import jax
import jax.numpy as jnp
from jax.experimental import pallas as pl

BATCH = 4
EMB_DIM = 1024
SEQ_LEN = 8192

D_BLK = 1024
S_BLK = 512


def _body(x_ref, pe_ref, o_ref):
    pe_t = jnp.transpose(pe_ref[...], (1, 0))
    o_ref[...] = x_ref[...] + pe_t[None, :, :]


def kernel(x, pe_table):
    grid = (EMB_DIM // D_BLK, SEQ_LEN // S_BLK)
    return pl.pallas_call(
        _body,
        grid=grid,
        in_specs=[
            pl.BlockSpec((BATCH, D_BLK, S_BLK), lambda i, j: (0, i, j)),
            pl.BlockSpec((S_BLK, D_BLK), lambda i, j: (j, i)),
        ],
        out_specs=pl.BlockSpec((BATCH, D_BLK, S_BLK), lambda i, j: (0, i, j)),
        out_shape=jax.ShapeDtypeStruct((BATCH, EMB_DIM, SEQ_LEN), jnp.float32),
    )(x, pe_table)
